# baseline (speedup 1.0000x reference)
import functools

import jax
import jax.numpy as jnp
from jax import lax
from jax.experimental import pallas as pl
from jax.experimental.pallas import tpu as pltpu, tpu_sc as plsc

_INFO = plsc.get_sparse_core_info()
_NC, _NS = _INFO.num_cores, _INFO.num_subcores
_NW = _NC * _NS
_L = 16
_BLK = 128
_G = _BLK // _L


@functools.partial(jax.jit, static_argnums=(2,))
def _sc_gather(idx_t, table2, items_per_w):
    J, I = idx_t.shape
    D = 64
    nci = I // _BLK
    mesh = plsc.VectorSubcoreMesh(core_axis_name="c", subcore_axis_name="s")

    vmem = lambda shape, dt: pltpu.VMEM(shape, dt)

    @functools.partial(
        pl.kernel,
        out_type=jax.ShapeDtypeStruct((J, D, I), jnp.float32),
        mesh=mesh,
        scratch_types=[
            vmem((_BLK,), jnp.int32),
            vmem((_BLK,), jnp.int32),
            vmem((_BLK,), jnp.int32),
            vmem((_BLK,), jnp.int32),
            vmem((_BLK,), jnp.int32),
            vmem((_BLK,), jnp.int32),
            vmem((_BLK, 128), jnp.float32),
            vmem((_BLK, 128), jnp.float32),
            vmem((D, _BLK), jnp.float32),
            vmem((D, _BLK), jnp.float32),
            vmem((_L, 17), jnp.float32),
            vmem((_L, 17), jnp.float32),
            pltpu.SemaphoreType.DMA,
            pltpu.SemaphoreType.DMA,
        ],
        compiler_params=pltpu.CompilerParams(
            use_tc_tiling_on_sc=True,
            needs_layout_passes=False,
            disable_bounds_checks=True,
        ),
    )
    def k(idx_hbm, tab_hbm, out_hbm,
          idx_a, idx_b, pidx_a, pidx_b, cb_a, cb_b,
          pairs_a, pairs_b, outt_a, outt_b, tr0_v, tr1_v, sem_a, sem_b):
        wid = lax.axis_index("s") * _NC + lax.axis_index("c")
        t_base = wid * items_per_w

        def stage(t, idx_v, pidx_v, cb_v, pairs_v, sem):
            j = t // nci
            c = t % nci
            pltpu.sync_copy(idx_hbm.at[j, pl.ds(c * _BLK, _BLK)], idx_v)
            for g in range(_G):
                v = idx_v[pl.ds(g * _L, _L)]
                pidx_v[pl.ds(g * _L, _L)] = lax.shift_right_logical(v, 1)
                cb_v[pl.ds(g * _L, _L)] = lax.mul(
                    lax.bitwise_and(v, 1), jnp.int32(D)
                )
            pltpu.async_copy(tab_hbm.at[pidx_v], pairs_v, sem)

        def drain(t, cb_v, pairs_v, outt_v, sem):
            pltpu.make_async_copy(
                tab_hbm.at[pl.ds(0, _BLK)], pairs_v, sem
            ).wait()
            iota = lax.iota(jnp.int32, _L)
            masks = [cb_v[pl.ds(g * _L, _L)] > 0 for g in range(_G)]

            def dgroup(dg, carry):
                dlo = dg * _L
                for sb in range(_G):
                    for si in range(_L):
                        s = sb * _L + si
                        tr0_v[si, pl.ds(0, _L)] = pairs_v[s, pl.ds(dlo, _L)]
                        tr1_v[si, pl.ds(0, _L)] = pairs_v[s, pl.ds(dlo + 64, _L)]
                    for di in range(_L):
                        dvec = jnp.full((_L,), di, jnp.int32)
                        v0 = plsc.load_gather(tr0_v, [iota, dvec])
                        v1 = plsc.load_gather(tr1_v, [iota, dvec])
                        outt_v[dlo + di, pl.ds(sb * _L, _L)] = jnp.where(
                            masks[sb], v1, v0
                        )
                return carry

            lax.fori_loop(0, D // _L, dgroup, 0)

            j = t // nci
            c = t % nci
            pltpu.sync_copy(outt_v, out_hbm.at[j, :, pl.ds(c * _BLK, _BLK)])

        stage(t_base, idx_a, pidx_a, cb_a, pairs_a, sem_a)

        def body(p, carry):
            t0 = t_base + 2 * p
            stage(t0 + 1, idx_b, pidx_b, cb_b, pairs_b, sem_b)
            drain(t0, cb_a, pairs_a, outt_a, sem_a)

            @pl.when(p < items_per_w // 2 - 1)
            def _():
                stage(t0 + 2, idx_a, pidx_a, cb_a, pairs_a, sem_a)

            drain(t0 + 1, cb_b, pairs_b, outt_b, sem_b)
            return carry

        lax.fori_loop(0, items_per_w // 2, body, 0)

    return k(idx_t, table2)


def kernel(location_x, loc_table, time_table, user_table):
    idx_t = location_x.T.astype(jnp.int32)
    table2 = loc_table.reshape(-1, 128)
    J, I = idx_t.shape
    items = J * (I // _BLK)
    out3 = _sc_gather(idx_t, table2, items // _NW)
    loc_emb = jnp.transpose(out3, (2, 0, 1))
    return (loc_emb, time_table, user_table)


# speedup vs baseline: 2.9096x; 2.9096x over previous
import functools

import jax
import jax.numpy as jnp
from jax import lax
from jax.experimental import pallas as pl
from jax.experimental.pallas import tpu as pltpu, tpu_sc as plsc

_INFO = plsc.get_sparse_core_info()
_NC, _NS = _INFO.num_cores, _INFO.num_subcores
_NW = _NC * _NS
_IW = 128
_K = 4
_CHUNK = _IW * _K


@functools.partial(jax.jit, static_argnums=(2,))
def _sc_gather(idx2d, table, num_chunks):
    D = table.shape[1]
    B = idx2d.shape[0] * idx2d.shape[1]
    rows_per_w = num_chunks * _K
    mesh = plsc.VectorSubcoreMesh(core_axis_name="c", subcore_axis_name="s")

    @functools.partial(
        pl.kernel,
        out_type=jax.ShapeDtypeStruct((B, D), jnp.float32),
        mesh=mesh,
        scratch_types=[
            pltpu.VMEM((_K, _IW), jnp.int32),
            pltpu.VMEM((_K, _IW), jnp.int32),
            pltpu.VMEM((_CHUNK, D), jnp.float32),
            pltpu.VMEM((_CHUNK, D), jnp.float32),
            pltpu.SemaphoreType.DMA,
            pltpu.SemaphoreType.DMA,
        ],
        compiler_params=pltpu.CompilerParams(use_tc_tiling_on_sc=False),
    )
    def k(idx_hbm, table_hbm, out_hbm, idx_a, idx_b, rows_a, rows_b, sem_a, sem_b):
        wid = lax.axis_index("s") * _NC + lax.axis_index("c")
        row_base = wid * rows_per_w

        def stage(idx_v, rows_v, sem, irow):
            pltpu.sync_copy(idx_hbm.at[pl.ds(irow, _K)], idx_v)
            for j in range(_K):
                pltpu.async_copy(
                    table_hbm.at[idx_v.at[j]],
                    rows_v.at[pl.ds(j * _IW, _IW)],
                    sem,
                )

        def drain(rows_v, sem, irow):
            for j in range(_K):
                pltpu.make_async_copy(
                    table_hbm.at[pl.ds(0, _IW)],
                    rows_v.at[pl.ds(j * _IW, _IW)],
                    sem,
                ).wait()
            pltpu.sync_copy(rows_v, out_hbm.at[pl.ds(irow * _IW, _CHUNK)])

        stage(idx_a, rows_a, sem_a, row_base)

        def body(p, carry):
            irow_a = row_base + 2 * p * _K
            stage(idx_b, rows_b, sem_b, irow_a + _K)
            drain(rows_a, sem_a, irow_a)

            @pl.when(p < num_chunks // 2 - 1)
            def _():
                stage(idx_a, rows_a, sem_a, irow_a + 2 * _K)

            drain(rows_b, sem_b, irow_a + _K)
            return carry

        lax.fori_loop(0, num_chunks // 2, body, 0)

    return k(idx2d, table)


def kernel(location_x, loc_table, time_table, user_table):
    orig_shape = location_x.shape
    idx_flat = location_x.reshape(-1).astype(jnp.int32)
    B = idx_flat.shape[0]
    b_per_w = B // _NW
    num_chunks = b_per_w // _CHUNK
    idx2d = idx_flat.reshape(B // _IW, _IW)
    out = _sc_gather(idx2d, loc_table, num_chunks)
    loc_emb = out.reshape(orig_shape + (loc_table.shape[1],))
    return (loc_emb, time_table, user_table)


# speedup vs baseline: 2.9223x; 1.0044x over previous
import functools

import jax
import jax.numpy as jnp
from jax import lax
from jax.experimental import pallas as pl
from jax.experimental.pallas import tpu as pltpu, tpu_sc as plsc

_INFO = plsc.get_sparse_core_info()
_NC, _NS = _INFO.num_cores, _INFO.num_subcores
_NW = _NC * _NS
_IW = 128
_K = 5
_CHUNK = _IW * _K


@functools.partial(jax.jit, static_argnums=(2,))
def _sc_gather(idx2d, table, num_chunks):
    D = table.shape[1]
    B = idx2d.shape[0] * idx2d.shape[1]
    rows_per_w = num_chunks * _K
    mesh = plsc.VectorSubcoreMesh(core_axis_name="c", subcore_axis_name="s")

    @functools.partial(
        pl.kernel,
        out_type=jax.ShapeDtypeStruct((B, D), jnp.float32),
        mesh=mesh,
        scratch_types=[
            pltpu.VMEM((_K, _IW), jnp.int32),
            pltpu.VMEM((_K, _IW), jnp.int32),
            pltpu.VMEM((_CHUNK, D), jnp.float32),
            pltpu.VMEM((_CHUNK, D), jnp.float32),
            pltpu.SemaphoreType.DMA,
            pltpu.SemaphoreType.DMA,
        ],
        compiler_params=pltpu.CompilerParams(use_tc_tiling_on_sc=False),
    )
    def k(idx_hbm, table_hbm, out_hbm, idx_a, idx_b, rows_a, rows_b, sem_a, sem_b):
        wid = lax.axis_index("s") * _NC + lax.axis_index("c")
        row_base = wid * rows_per_w

        def stage(idx_v, rows_v, sem, irow):
            pltpu.sync_copy(idx_hbm.at[pl.ds(irow, _K)], idx_v)
            for j in range(_K):
                pltpu.async_copy(
                    table_hbm.at[idx_v.at[j]],
                    rows_v.at[pl.ds(j * _IW, _IW)],
                    sem,
                )

        def drain(rows_v, sem, irow):
            for j in range(_K):
                pltpu.make_async_copy(
                    table_hbm.at[pl.ds(0, _IW)],
                    rows_v.at[pl.ds(j * _IW, _IW)],
                    sem,
                ).wait()
            pltpu.sync_copy(rows_v, out_hbm.at[pl.ds(irow * _IW, _CHUNK)])

        stage(idx_a, rows_a, sem_a, row_base)

        def body(p, carry):
            irow_a = row_base + 2 * p * _K
            stage(idx_b, rows_b, sem_b, irow_a + _K)
            drain(rows_a, sem_a, irow_a)

            @pl.when(p < num_chunks // 2 - 1)
            def _():
                stage(idx_a, rows_a, sem_a, irow_a + 2 * _K)

            drain(rows_b, sem_b, irow_a + _K)
            return carry

        lax.fori_loop(0, num_chunks // 2, body, 0)

    return k(idx2d, table)


def kernel(location_x, loc_table, time_table, user_table):
    orig_shape = location_x.shape
    idx_flat = location_x.reshape(-1).astype(jnp.int32)
    B = idx_flat.shape[0]
    b_per_w = B // _NW
    num_chunks = b_per_w // _CHUNK
    idx2d = idx_flat.reshape(B // _IW, _IW)
    out = _sc_gather(idx2d, loc_table, num_chunks)
    loc_emb = out.reshape(orig_shape + (loc_table.shape[1],))
    return (loc_emb, time_table, user_table)
